# Initial kernel scaffold; baseline (speedup 1.0000x reference)
import jax
import jax.numpy as jnp
from jax.experimental import pallas as pl


def kernel(x, gamma, beta, w, b):
    raise NotImplementedError("write your pallas kernel here")


# speedup vs baseline: 4.4901x; 4.4901x over previous
import functools

import jax
import jax.numpy as jnp
from jax import lax
from jax.experimental import pallas as pl
from jax.experimental.pallas import tpu as pltpu


def _round_up(x, m):
    return ((x + m - 1) // m) * m


def _prenorm_matmul_kernel(x_ref, g_ref, b_ref, w_ref, bias_ref, o_ref,
                           *, eps, true_dim):
    x = x_ref[...].astype(jnp.float32)
    kp = x.shape[-1]
    inv_d = 1.0 / float(true_dim)
    mean = jnp.sum(x, axis=-1, keepdims=True) * inv_d
    xc = x - mean
    if kp != true_dim:
        mask = lax.broadcasted_iota(jnp.int32, (1, kp), 1) < true_dim
        xc = jnp.where(mask, xc, 0.0)
    var = jnp.sum(xc * xc, axis=-1, keepdims=True) * inv_d
    inv = lax.rsqrt(var + eps)
    y = xc * inv * g_ref[...].astype(jnp.float32) + b_ref[...].astype(jnp.float32)
    y = y.astype(jnp.bfloat16)
    acc = jnp.dot(y, w_ref[...], preferred_element_type=jnp.float32)
    o_ref[...] = (acc + bias_ref[...].astype(jnp.float32)).astype(o_ref.dtype)


def kernel(x, gamma, beta, w, b):
    eps = 1e-5
    orig_shape = x.shape
    din = orig_shape[-1]
    dout = w.shape[1]
    x2 = x.reshape(-1, din)
    rows = x2.shape[0]

    kp = max(_round_up(din, 128), 128)
    np_ = max(_round_up(dout, 128), 128)

    tm = min(512, _round_up(rows, 8))
    rows_p = _round_up(rows, tm)

    x_p = jnp.pad(x2, ((0, rows_p - rows), (0, kp - din)))
    g_p = jnp.pad(gamma.reshape(1, din), ((0, 0), (0, kp - din)))
    b_p = jnp.pad(beta.reshape(1, din), ((0, 0), (0, kp - din)))
    w_p = jnp.pad(w, ((0, kp - din), (0, np_ - dout))).astype(jnp.bfloat16)
    bias_p = jnp.pad(b.reshape(1, dout), ((0, 0), (0, np_ - dout)))

    cost = pl.CostEstimate(
        flops=2 * rows_p * kp * np_ + 8 * rows_p * kp,
        transcendentals=rows_p,
        bytes_accessed=rows_p * kp * 4 + kp * np_ * 2 + rows_p * np_ * 4,
    )

    out = pl.pallas_call(
        functools.partial(_prenorm_matmul_kernel, eps=eps, true_dim=din),
        out_shape=jax.ShapeDtypeStruct((rows_p, np_), x.dtype),
        grid_spec=pltpu.PrefetchScalarGridSpec(
            num_scalar_prefetch=0,
            grid=(rows_p // tm,),
            in_specs=[
                pl.BlockSpec((tm, kp), lambda i: (i, 0)),
                pl.BlockSpec((1, kp), lambda i: (0, 0)),
                pl.BlockSpec((1, kp), lambda i: (0, 0)),
                pl.BlockSpec((kp, np_), lambda i: (0, 0)),
                pl.BlockSpec((1, np_), lambda i: (0, 0)),
            ],
            out_specs=pl.BlockSpec((tm, np_), lambda i: (i, 0)),
        ),
        compiler_params=pltpu.CompilerParams(
            dimension_semantics=("parallel",),
            vmem_limit_bytes=60 * 1024 * 1024,
        ),
        cost_estimate=cost,
    )(x_p, g_p, b_p, w_p, bias_p)
    return out[:rows, :dout].reshape(orig_shape[:-1] + (dout,))


# speedup vs baseline: 5.1354x; 1.1437x over previous
import functools

import jax
import jax.numpy as jnp
from jax import lax
from jax.experimental import pallas as pl
from jax.experimental.pallas import tpu as pltpu


def _round_up(x, m):
    return ((x + m - 1) // m) * m


def _prenorm_matmul_kernel(x_ref, g_ref, b_ref, w_ref, bias_ref, o_ref,
                           *, eps, true_dim):
    x = x_ref[...].astype(jnp.float32)
    kp = x.shape[-1]
    inv_d = 1.0 / float(true_dim)
    mean = jnp.sum(x, axis=-1, keepdims=True) * inv_d
    xc = x - mean
    if kp != true_dim:
        mask = lax.broadcasted_iota(jnp.int32, (1, kp), 1) < true_dim
        xc = jnp.where(mask, xc, 0.0)
    var = jnp.sum(xc * xc, axis=-1, keepdims=True) * inv_d
    inv = lax.rsqrt(var + eps)
    y = xc * inv * g_ref[...].astype(jnp.float32) + b_ref[...].astype(jnp.float32)
    y = y.astype(jnp.bfloat16)
    acc = jnp.dot(y, w_ref[...].astype(jnp.bfloat16),
                  preferred_element_type=jnp.float32)
    o_ref[...] = (acc + bias_ref[...].astype(jnp.float32)).astype(o_ref.dtype)


def kernel(x, gamma, beta, w, b):
    eps = 1e-5
    orig_shape = x.shape
    din = orig_shape[-1]
    dout = w.shape[1]
    x2 = x.reshape(-1, din)
    rows = x2.shape[0]

    kp = max(_round_up(din, 128), 128)
    np_ = max(_round_up(dout, 128), 128)

    tm = min(512, _round_up(rows, 8))
    rows_p = _round_up(rows, tm)

    x_p = jnp.pad(x2, ((0, rows_p - rows), (0, kp - din)))
    g_p = jnp.pad(gamma.reshape(1, din), ((0, 0), (0, kp - din)))
    b_p = jnp.pad(beta.reshape(1, din), ((0, 0), (0, kp - din)))
    w_p = jnp.pad(w, ((0, kp - din), (0, np_ - dout)))
    bias_p = jnp.pad(b.reshape(1, dout), ((0, 0), (0, np_ - dout)))

    cost = pl.CostEstimate(
        flops=2 * rows_p * kp * np_ + 8 * rows_p * kp,
        transcendentals=rows_p,
        bytes_accessed=rows_p * kp * 4 + kp * np_ * 2 + rows_p * np_ * 4,
    )

    out = pl.pallas_call(
        functools.partial(_prenorm_matmul_kernel, eps=eps, true_dim=din),
        out_shape=jax.ShapeDtypeStruct((rows_p, np_), x.dtype),
        grid_spec=pltpu.PrefetchScalarGridSpec(
            num_scalar_prefetch=0,
            grid=(rows_p // tm,),
            in_specs=[
                pl.BlockSpec((tm, kp), lambda i: (i, 0)),
                pl.BlockSpec((1, kp), lambda i: (0, 0)),
                pl.BlockSpec((1, kp), lambda i: (0, 0)),
                pl.BlockSpec((kp, np_), lambda i: (0, 0)),
                pl.BlockSpec((1, np_), lambda i: (0, 0)),
            ],
            out_specs=pl.BlockSpec((tm, np_), lambda i: (i, 0)),
        ),
        compiler_params=pltpu.CompilerParams(
            dimension_semantics=("parallel",),
            vmem_limit_bytes=60 * 1024 * 1024,
        ),
        cost_estimate=cost,
    )(x_p, g_p, b_p, w_p, bias_p)
    return out[:rows, :dout].reshape(orig_shape[:-1] + (dout,))
